# baseline (speedup 1.0000x reference)
import jax
import jax.numpy as jnp
import numpy as np
from jax.experimental import pallas as pl
from jax.experimental.pallas import tpu as pltpu

_NL = 8
_LEAF = 128
_HID = _NL * _LEAF
_LATENT = 384
_SEQ = 196
_IMB = 4


def _tree_consts():
    C = np.zeros((32, _NL), np.float32)
    E = np.zeros((_NL, _HID), np.float32)
    for l in range(_NL):
        b2, b1, b0 = (l >> 2) & 1, (l >> 1) & 1, l & 1
        C[0 + 0, l] = 1.0 if b2 else -1.0
        C[8 + 1 + b2, l] = 1.0 if b1 else -1.0
        C[16 + 3 + (l >> 1), l] = 1.0 if b0 else -1.0
        C[24, l] = 0.0 if b2 else 1.0
        C[25, l] = 0.0 if b1 else 1.0
        C[26, l] = 0.0 if b0 else 1.0
        E[l, l * _LEAF:(l + 1) * _LEAF] = 1.0
    return jnp.asarray(C), jnp.asarray(E)


def _dot16(a, b):
    return jnp.dot(a.astype(jnp.bfloat16), b, preferred_element_type=jnp.float32)


def _fff(x, tree, E, nwT, nb, W1, b1, W2, b2, act):
    e = jax.nn.sigmoid(_dot16(x, nwT) + nb)
    t0 = jnp.dot(e, tree[0:8]) + tree[24:25]
    t1 = jnp.dot(e, tree[8:16]) + tree[25:26]
    t2 = jnp.dot(e, tree[16:24]) + tree[26:27]
    mix = t0 * t1 * t2
    h = act(_dot16(x, W1) + b1)
    hm = h.astype(jnp.bfloat16) * jnp.dot(mix, E).astype(jnp.bfloat16)
    return jnp.dot(hm, W2, preferred_element_type=jnp.float32) + jnp.dot(mix, b2)


def _gelu(z):
    return 0.5 * z * (1.0 + jax.lax.erf(z * (2.0 ** -0.5)))


def _layernorm(x):
    mean = jnp.mean(x, axis=-1, keepdims=True)
    d = x - mean
    var = jnp.sum(d * d, axis=-1, keepdims=True) / (_LATENT - 1)
    std = jnp.sqrt(var)
    return d / jnp.sqrt(std + 1e-05)


def _mega_body(img_ref, pe_ref, tree_ref, E_ref,
               tnwT_ref, tnb_ref, tW1_ref, tb1_ref, tW2_ref, tb2_ref,
               inwT_ref, inb_ref, iW1_ref, ib1_ref, iW2_ref, ib2_ref,
               out_ref):
    tree = tree_ref[...]
    E = E_ref[...]

    def fff_j(z, j, act):
        return _fff(z, tree, E, inwT_ref[j], inb_ref[j], iW1_ref[j],
                    ib1_ref[j], iW2_ref[j], ib2_ref[j], act)

    X = img_ref[...]
    P = [X[i].reshape(3, 14, 16, 14, 16).transpose(1, 3, 0, 2, 4)
         .reshape(_SEQ, 768).astype(jnp.bfloat16) for i in range(_IMB)]
    P = jnp.concatenate(P, axis=0)
    x = _fff(P, tree, E, tnwT_ref[...], tnb_ref[...], tW1_ref[...],
             tb1_ref[...], tW2_ref[...], tb2_ref[...], jax.nn.relu)
    x = x + pe_ref[...]
    for base in (0, 4):
        xn = _layernorm(x)
        q = fff_j(xn, base + 0, jax.nn.relu).astype(jnp.bfloat16)
        k = fff_j(xn, base + 1, jax.nn.relu).astype(jnp.bfloat16)
        v = fff_j(xn, base + 2, jax.nn.relu).astype(jnp.bfloat16)
        avs = []
        for i in range(_IMB):
            sl = slice(i * _SEQ, (i + 1) * _SEQ)
            s = jax.lax.dot_general(q[sl], k[sl], (((1,), (1,)), ((), ())),
                                    preferred_element_type=jnp.float32)
            s = s / (_LATENT ** 0.5)
            m = jnp.max(s, axis=-1, keepdims=True)
            p = jnp.exp(s - m)
            att = p / jnp.sum(p, axis=-1, keepdims=True)
            avs.append(_dot16(att, v[sl]))
        x1 = xn + jnp.concatenate(avs, axis=0)
        x = x1 + fff_j(x1, base + 3, _gelu)
    out_ref[...] = jnp.mean(x.reshape(_IMB, _SEQ, _LATENT), axis=1,
                            keepdims=True)


def _head_body(x_ref, tree_ref, E_ref, nwT_ref, nb_ref, W1_ref, b1_ref,
               W2_ref, b2_ref, out_ref):
    out_ref[...] = _fff(x_ref[...], tree_ref[...], E_ref[...], nwT_ref[...],
                        nb_ref[...], W1_ref[...], b1_ref[...], W2_ref[...],
                        b2_ref[...], jax.nn.relu)


def _full(arr):
    nd = len(arr.shape)
    return pl.BlockSpec(arr.shape, lambda i, _nd=nd: (0,) * _nd)


def _prep_fff(nw, nb, w1, b1, w2, b2, in_w):
    nwT = jnp.swapaxes(jnp.pad(nw, [(0, 0)] * (nw.ndim - 2) + [(0, 1), (0, 0)]),
                       -1, -2)
    nbr = jnp.swapaxes(jnp.pad(nb, [(0, 0)] * (nb.ndim - 2) + [(0, 1), (0, 0)]),
                       -1, -2)
    W1 = jnp.swapaxes(w1, -3, -2).reshape(w1.shape[:-3] + (in_w, _HID))
    b1r = b1.reshape(b1.shape[:-2] + (1, _HID))
    W2 = w2.reshape(w2.shape[:-3] + (_HID, w2.shape[-1]))
    return (nwT.astype(jnp.bfloat16), nbr, W1.astype(jnp.bfloat16), b1r,
            W2.astype(jnp.bfloat16), b2)


def kernel(imgs, tok_nw, tok_nb, tok_w1, tok_b1, tok_w2, tok_b2,
           inn_nw, inn_nb, inn_w1, inn_b1, inn_w2, inn_b2,
           out_nw, out_nb, out_w1, out_b1, out_w2, out_b2):
    B, C, H, W = imgs.shape
    in_w = C * 16 * 16

    tree, E = _tree_consts()

    s = jnp.arange(_SEQ, dtype=jnp.float32)
    pe = jnp.where((jnp.arange(_LATENT) % 2 == 0)[None, :],
                   jnp.sin(s)[:, None], jnp.cos(s)[:, None])

    tok = _prep_fff(tok_nw, tok_nb, tok_w1, tok_b1, tok_w2, tok_b2, in_w)
    inn = _prep_fff(inn_nw, inn_nb, inn_w1, inn_b1, inn_w2, inn_b2, _LATENT)
    out = _prep_fff(out_nw, out_nb, out_w1, out_b1, out_w2, out_b2, _LATENT)

    pe4 = jnp.tile(pe, (_IMB, 1))
    xm = pl.pallas_call(
        _mega_body,
        grid=(B // _IMB,),
        in_specs=[pl.BlockSpec((_IMB, C, H, W), lambda i: (i, 0, 0, 0)),
                  _full(pe4), _full(tree), _full(E)] +
                 [_full(a) for a in tok] + [_full(a) for a in inn],
        out_specs=pl.BlockSpec((_IMB, 1, _LATENT), lambda i: (i, 0, 0)),
        out_shape=jax.ShapeDtypeStruct((B, 1, _LATENT), jnp.float32),
        compiler_params=pltpu.CompilerParams(
            dimension_semantics=("parallel",)),
    )(imgs, pe4, tree, E, *tok, *inn)
    xm = xm.reshape(B, _LATENT)

    y = pl.pallas_call(
        _head_body,
        grid=(1,),
        in_specs=[_full(xm), _full(tree), _full(E)] + [_full(a) for a in out],
        out_specs=_full(jax.ShapeDtypeStruct((B, out_w2.shape[-1]), jnp.float32)),
        out_shape=jax.ShapeDtypeStruct((B, out_w2.shape[-1]), jnp.float32),
    )(xm, tree, E, *out)
    return y


# speedup vs baseline: 1.0312x; 1.0312x over previous
import jax
import jax.numpy as jnp
import numpy as np
from jax.experimental import pallas as pl
from jax.experimental.pallas import tpu as pltpu

_NL = 8
_LEAF = 128
_HID = _NL * _LEAF
_LATENT = 384
_SEQ = 196
_IMB = 1


def _tree_consts():
    C = np.zeros((32, _NL), np.float32)
    E = np.zeros((_NL, _HID), np.float32)
    for l in range(_NL):
        b2, b1, b0 = (l >> 2) & 1, (l >> 1) & 1, l & 1
        C[0 + 0, l] = 1.0 if b2 else -1.0
        C[8 + 1 + b2, l] = 1.0 if b1 else -1.0
        C[16 + 3 + (l >> 1), l] = 1.0 if b0 else -1.0
        C[24, l] = 0.0 if b2 else 1.0
        C[25, l] = 0.0 if b1 else 1.0
        C[26, l] = 0.0 if b0 else 1.0
        E[l, l * _LEAF:(l + 1) * _LEAF] = 1.0
    return jnp.asarray(C), jnp.asarray(E)


def _dot16(a, b):
    return jnp.dot(a.astype(jnp.bfloat16), b, preferred_element_type=jnp.float32)


def _fff(x, tree, E, nwT, nb, W1, b1, W2, b2, act):
    e = jax.nn.sigmoid(_dot16(x, nwT) + nb)
    t0 = jnp.dot(e, tree[0:8]) + tree[24:25]
    t1 = jnp.dot(e, tree[8:16]) + tree[25:26]
    t2 = jnp.dot(e, tree[16:24]) + tree[26:27]
    mix = t0 * t1 * t2
    h = act(_dot16(x, W1) + b1)
    hm = h.astype(jnp.bfloat16) * jnp.dot(mix, E).astype(jnp.bfloat16)
    return jnp.dot(hm, W2, preferred_element_type=jnp.float32) + jnp.dot(mix, b2)


def _gelu(z):
    return 0.5 * z * (1.0 + jax.lax.erf(z * (2.0 ** -0.5)))


def _layernorm(x):
    mean = jnp.mean(x, axis=-1, keepdims=True)
    d = x - mean
    var = jnp.sum(d * d, axis=-1, keepdims=True) / (_LATENT - 1)
    std = jnp.sqrt(var)
    return d / jnp.sqrt(std + 1e-05)


def _mega_body(img_ref, pe_ref, tree_ref, E_ref,
               tnwT_ref, tnb_ref, tW1_ref, tb1_ref, tW2_ref, tb2_ref,
               inwT_ref, inb_ref, iW1_ref, ib1_ref, iW2_ref, ib2_ref,
               out_ref):
    tree = tree_ref[...]
    E = E_ref[...]

    def fff_j(z, j, act):
        return _fff(z, tree, E, inwT_ref[j], inb_ref[j], iW1_ref[j],
                    ib1_ref[j], iW2_ref[j], ib2_ref[j], act)

    X = img_ref[...]
    P = [X[i].reshape(3, 14, 16, 14, 16).transpose(1, 3, 0, 2, 4)
         .reshape(_SEQ, 768).astype(jnp.bfloat16) for i in range(_IMB)]
    P = jnp.concatenate(P, axis=0)
    x = _fff(P, tree, E, tnwT_ref[...], tnb_ref[...], tW1_ref[...],
             tb1_ref[...], tW2_ref[...], tb2_ref[...], jax.nn.relu)
    x = x + pe_ref[...]
    for base in (0, 4):
        xn = _layernorm(x)
        q = fff_j(xn, base + 0, jax.nn.relu).astype(jnp.bfloat16)
        k = fff_j(xn, base + 1, jax.nn.relu).astype(jnp.bfloat16)
        v = fff_j(xn, base + 2, jax.nn.relu).astype(jnp.bfloat16)
        avs = []
        for i in range(_IMB):
            sl = slice(i * _SEQ, (i + 1) * _SEQ)
            s = jax.lax.dot_general(q[sl], k[sl], (((1,), (1,)), ((), ())),
                                    preferred_element_type=jnp.float32)
            s = s / (_LATENT ** 0.5)
            m = jnp.max(s, axis=-1, keepdims=True)
            p = jnp.exp(s - m)
            att = p / jnp.sum(p, axis=-1, keepdims=True)
            avs.append(_dot16(att, v[sl]))
        x1 = xn + jnp.concatenate(avs, axis=0)
        x = x1 + fff_j(x1, base + 3, _gelu)
    out_ref[...] = jnp.mean(x.reshape(_IMB, _SEQ, _LATENT), axis=1,
                            keepdims=True)


def _head_body(x_ref, tree_ref, E_ref, nwT_ref, nb_ref, W1_ref, b1_ref,
               W2_ref, b2_ref, out_ref):
    out_ref[...] = _fff(x_ref[...], tree_ref[...], E_ref[...], nwT_ref[...],
                        nb_ref[...], W1_ref[...], b1_ref[...], W2_ref[...],
                        b2_ref[...], jax.nn.relu)


def _full(arr):
    nd = len(arr.shape)
    return pl.BlockSpec(arr.shape, lambda i, _nd=nd: (0,) * _nd)


def _prep_fff(nw, nb, w1, b1, w2, b2, in_w):
    nwT = jnp.swapaxes(jnp.pad(nw, [(0, 0)] * (nw.ndim - 2) + [(0, 1), (0, 0)]),
                       -1, -2)
    nbr = jnp.swapaxes(jnp.pad(nb, [(0, 0)] * (nb.ndim - 2) + [(0, 1), (0, 0)]),
                       -1, -2)
    W1 = jnp.swapaxes(w1, -3, -2).reshape(w1.shape[:-3] + (in_w, _HID))
    b1r = b1.reshape(b1.shape[:-2] + (1, _HID))
    W2 = w2.reshape(w2.shape[:-3] + (_HID, w2.shape[-1]))
    return (nwT.astype(jnp.bfloat16), nbr, W1.astype(jnp.bfloat16), b1r,
            W2.astype(jnp.bfloat16), b2)


def kernel(imgs, tok_nw, tok_nb, tok_w1, tok_b1, tok_w2, tok_b2,
           inn_nw, inn_nb, inn_w1, inn_b1, inn_w2, inn_b2,
           out_nw, out_nb, out_w1, out_b1, out_w2, out_b2):
    B, C, H, W = imgs.shape
    in_w = C * 16 * 16

    tree, E = _tree_consts()

    s = jnp.arange(_SEQ, dtype=jnp.float32)
    pe = jnp.where((jnp.arange(_LATENT) % 2 == 0)[None, :],
                   jnp.sin(s)[:, None], jnp.cos(s)[:, None])

    tok = _prep_fff(tok_nw, tok_nb, tok_w1, tok_b1, tok_w2, tok_b2, in_w)
    inn = _prep_fff(inn_nw, inn_nb, inn_w1, inn_b1, inn_w2, inn_b2, _LATENT)
    out = _prep_fff(out_nw, out_nb, out_w1, out_b1, out_w2, out_b2, _LATENT)

    pe4 = jnp.tile(pe, (_IMB, 1))
    xm = pl.pallas_call(
        _mega_body,
        grid=(B // _IMB,),
        in_specs=[pl.BlockSpec((_IMB, C, H, W), lambda i: (i, 0, 0, 0)),
                  _full(pe4), _full(tree), _full(E)] +
                 [_full(a) for a in tok] + [_full(a) for a in inn],
        out_specs=pl.BlockSpec((_IMB, 1, _LATENT), lambda i: (i, 0, 0)),
        out_shape=jax.ShapeDtypeStruct((B, 1, _LATENT), jnp.float32),
        compiler_params=pltpu.CompilerParams(
            dimension_semantics=("parallel",)),
    )(imgs, pe4, tree, E, *tok, *inn)
    xm = xm.reshape(B, _LATENT)

    y = pl.pallas_call(
        _head_body,
        grid=(1,),
        in_specs=[_full(xm), _full(tree), _full(E)] + [_full(a) for a in out],
        out_specs=_full(jax.ShapeDtypeStruct((B, out_w2.shape[-1]), jnp.float32)),
        out_shape=jax.ShapeDtypeStruct((B, out_w2.shape[-1]), jnp.float32),
    )(xm, tree, E, *out)
    return y
